# Initial kernel scaffold; baseline (speedup 1.0000x reference)
import jax
import jax.numpy as jnp
from jax.experimental import pallas as pl


def kernel(x, gate_w, gate_b, w1, b1, w2, b2):
    raise NotImplementedError("write your pallas kernel here")


# speedup vs baseline: 1.2122x; 1.2122x over previous
import functools

import jax
import jax.numpy as jnp
from jax.experimental import pallas as pl
from jax.experimental.pallas import tpu as pltpu

EMB = 1024
NUM_EXPERTS = 8
TOP_K = 2
HID = 2048
B, S = 2, 2048
T = B * S
EPAD = 128
LOAD_COEFF = 0.1
Z_ROUTER_COEFF = 0.001

_SQRT_2_OVER_PI = 0.7978845608028654


def _gelu_tanh(x):
    return 0.5 * x * (1.0 + jnp.tanh(_SQRT_2_OVER_PI * (x + 0.044715 * x * x * x)))


def _router_body(x_ref, gw_ref, gb_ref, wgt_ref, loss_ref):
    x = x_ref[...]
    logits = jax.lax.dot_general(
        x, gw_ref[...], (((1,), (0,)), ((), ())),
        preferred_element_type=jnp.float32) + gb_ref[...]
    lane = jax.lax.broadcasted_iota(jnp.int32, (T, EPAD), 1)
    valid = lane < NUM_EXPERTS
    lm = jnp.where(valid, logits, -1e30)
    m = jnp.max(lm, axis=1, keepdims=True)
    ex = jnp.where(valid, jnp.exp(lm - m), 0.0)
    denom = jnp.sum(ex, axis=1, keepdims=True)
    probs = ex / denom
    lse = m + jnp.log(denom)

    p1 = jnp.max(probs, axis=1, keepdims=True)
    i1 = jnp.min(jnp.where(probs == p1, lane, EPAD), axis=1, keepdims=True)
    mask1 = lane == i1
    p2 = jnp.max(jnp.where(mask1, -1.0, probs), axis=1, keepdims=True)
    i2 = jnp.min(jnp.where((probs == p2) & (~mask1), lane, EPAD),
                 axis=1, keepdims=True)
    ssum = p1 + p2
    w1p = p1 / ssum
    w2p = p2 / ssum
    onehot = (mask1).astype(jnp.float32) + (lane == i2).astype(jnp.float32)
    wgt_ref[...] = jnp.where(mask1, w1p, 0.0) + jnp.where(lane == i2, w2p, 0.0)

    z_loss = jnp.sum(lse * lse) * (1.0 / T)
    counts = jnp.sum(onehot, axis=0, keepdims=True)
    p_mean = jnp.sum(probs, axis=0, keepdims=True) * (1.0 / T)
    f_i = counts * (1.0 / (TOP_K * T))
    load_loss = NUM_EXPERTS * jnp.sum(f_i * p_mean)
    loss_ref[0, 0] = Z_ROUTER_COEFF * z_loss + LOAD_COEFF * load_loss


def _router(x2, gw_pad, gb_pad):
    return pl.pallas_call(
        _router_body,
        out_shape=(
            jax.ShapeDtypeStruct((T, EPAD), jnp.float32),
            jax.ShapeDtypeStruct((1, 1), jnp.float32),
        ),
        in_specs=[
            pl.BlockSpec((T, EMB), lambda: (0, 0)),
            pl.BlockSpec((EMB, EPAD), lambda: (0, 0)),
            pl.BlockSpec((1, EPAD), lambda: (0, 0)),
        ],
        out_specs=(
            pl.BlockSpec((T, EPAD), lambda: (0, 0)),
            pl.BlockSpec(memory_space=pltpu.SMEM),
        ),
    )(x2, gw_pad, gb_pad)


BT = 512
NT = T // BT


def _ffn_body(x_ref, w1_ref, b1_ref, w2_ref, b2_ref, wgt_ref, out_ref):
    e = pl.program_id(0)
    t = pl.program_id(1)
    xb = x_ref[...]
    h = jax.lax.dot_general(
        xb, w1_ref[0], (((1,), (0,)), ((), ())),
        preferred_element_type=jnp.float32) + b1_ref[0]
    h = _gelu_tanh(h)
    y = jax.lax.dot_general(
        h, w2_ref[0], (((1,), (0,)), ((), ())),
        preferred_element_type=jnp.float32) + b2_ref[0]
    lane = jax.lax.broadcasted_iota(jnp.int32, (BT, EPAD), 1)
    w = jnp.sum(jnp.where(lane == e, wgt_ref[...], 0.0), axis=1, keepdims=True)
    contrib = y * w
    row0 = t * BT

    @pl.when(e == 0)
    def _():
        out_ref[pl.ds(row0, BT), :] = contrib

    @pl.when(e > 0)
    def _():
        out_ref[pl.ds(row0, BT), :] += contrib


def _ffn(x2, w1, b1, w2, b2, wgt):
    return pl.pallas_call(
        _ffn_body,
        grid=(NUM_EXPERTS, NT),
        in_specs=[
            pl.BlockSpec((BT, EMB), lambda e, t: (t, 0)),
            pl.BlockSpec((1, EMB, HID), lambda e, t: (e, 0, 0)),
            pl.BlockSpec((1, 1, HID), lambda e, t: (e, 0, 0)),
            pl.BlockSpec((1, HID, EMB), lambda e, t: (e, 0, 0)),
            pl.BlockSpec((1, 1, EMB), lambda e, t: (e, 0, 0)),
            pl.BlockSpec((BT, EPAD), lambda e, t: (t, 0)),
        ],
        out_specs=pl.BlockSpec((T, EMB), lambda e, t: (0, 0)),
        out_shape=jax.ShapeDtypeStruct((T, EMB), jnp.float32),
        compiler_params=pltpu.CompilerParams(
            dimension_semantics=("arbitrary", "arbitrary")),
    )(x2, w1, b1, w2, b2, wgt)


@jax.jit
def kernel(x, gate_w, gate_b, w1, b1, w2, b2):
    x2 = x.reshape(T, EMB)
    gw_pad = jnp.pad(gate_w, ((0, 0), (0, EPAD - NUM_EXPERTS)))
    gb_pad = jnp.pad(gate_b, (0, EPAD - NUM_EXPERTS)).reshape(1, EPAD)
    wgt, loss = _router(x2, gw_pad, gb_pad)
    out = _ffn(x2, w1, b1.reshape(NUM_EXPERTS, 1, HID),
               w2, b2.reshape(NUM_EXPERTS, 1, EMB), wgt)
    return out.reshape(B, S, EMB), loss[0, 0]
